# baseline (speedup 1.0000x reference)
import functools

import jax
import jax.numpy as jnp
from jax.experimental import pallas as pl
from jax.experimental.pallas import tpu as pltpu
from jax.experimental.pallas import tpu_sc as plsc

N = 50000
E = 800000
L = 3200000
NUM_ELEMENTS = 92
FEAT_DIM = 92

D = 4
_RPF = 128 // D
_C3 = 3 * 128 // D

_G1_WIN = 2560
_G2_WIN = 3200


def _sc_gather(table, idx_flat, num_idx, window):
    mesh = plsc.VectorSubcoreMesh(core_axis_name="c", subcore_axis_name="s")

    @functools.partial(
        pl.kernel,
        out_type=jax.ShapeDtypeStruct((num_idx, D), jnp.float32),
        mesh=mesh,
        compiler_params=pltpu.CompilerParams(use_tc_tiling_on_sc=False),
    )
    def gather_kernel(tab_hbm, idx_hbm, out_hbm):
        def body(idx_vmem, out_vmem):
            pltpu.sync_copy(tab_hbm.at[idx_vmem.at[0]], out_vmem)

        pltpu.emit_pipeline(
            body,
            grid=(num_idx // window,),
            in_specs=[pl.BlockSpec((1, window), lambda i: (0, i))],
            out_specs=[pl.BlockSpec((window, D), lambda i: (i, 0))],
            core_axis_name=("c", "s"),
            dimension_semantics=(pltpu.PARALLEL,),
        )(idx_hbm, out_hbm)

    return gather_kernel(table, idx_flat)


def _edge_diff_kernel(src_ref, dst_ref, r8_ref, r3_ref):
    diff = dst_ref[...] - src_ref[...]
    r8_ref[...] = diff
    i = jax.lax.broadcasted_iota(jnp.int32, (128, _C3), 0)
    j = jax.lax.broadcasted_iota(jnp.int32, (128, _C3), 1)
    m = ((i % D < 3) & (j == 3 * (i // D) + (i % D))).astype(jnp.float32)
    r3_ref[...] = jnp.dot(diff, m, preferred_element_type=jnp.float32)


def _cosine_kernel(a_ref, b_ref, out_ref):
    a = a_ref[...]
    b = b_ref[...]
    i = jax.lax.broadcasted_iota(jnp.int32, (128, _RPF), 0)
    j = jax.lax.broadcasted_iota(jnp.int32, (128, _RPF), 1)
    m = (i // D == j).astype(jnp.float32)
    num = -jnp.dot(a * b, m, preferred_element_type=jnp.float32)
    s1 = jnp.dot(a * a, m, preferred_element_type=jnp.float32)
    s2 = jnp.dot(b * b, m, preferred_element_type=jnp.float32)
    denom = jnp.sqrt(s1 * s2)
    out_ref[...] = jnp.clip(num / (denom + 1e-12), -1.0, 1.0)


def _embed_kernel(an_ref, tab_ref, out_ref):
    an = an_ref[...]
    iota = jax.lax.broadcasted_iota(jnp.int32, (an.shape[0], 128), 1)
    oh = (an == iota).astype(jnp.float32)
    res = jnp.dot(oh, tab_ref[...], preferred_element_type=jnp.float32)
    out_ref[...] = res[:, :FEAT_DIM]


def kernel(atomic_number, positions, edge_index, lg_pairs, atom_table):
    pos8 = jnp.pad(positions, ((0, 0), (0, D - 3)))
    eidx_flat = edge_index.reshape(1, 2 * E)
    lg_flat = lg_pairs.reshape(1, 2 * L)
    tab128 = jnp.pad(atom_table, ((0, 128 - NUM_ELEMENTS), (0, 128 - FEAT_DIM)))
    an_col = atomic_number.reshape(N, 1).astype(jnp.int32)

    g1 = _sc_gather(pos8, eidx_flat, 2 * E, _G1_WIN)
    g1_flat = g1.reshape(2 * E * D // 128, 128)

    n_half1 = E * D // 128
    b1 = 1000
    r8_flat, r3_flat = pl.pallas_call(
        _edge_diff_kernel,
        grid=(n_half1 // b1,),
        in_specs=[
            pl.BlockSpec((b1, 128), lambda i: (i, 0)),
            pl.BlockSpec((b1, 128), lambda i: (i + n_half1 // b1, 0)),
        ],
        out_specs=[
            pl.BlockSpec((b1, 128), lambda i: (i, 0)),
            pl.BlockSpec((b1, _C3), lambda i: (i, 0)),
        ],
        out_shape=[
            jax.ShapeDtypeStruct((n_half1, 128), jnp.float32),
            jax.ShapeDtypeStruct((n_half1, _C3), jnp.float32),
        ],
    )(g1_flat, g1_flat)
    r8 = r8_flat.reshape(E, D)
    r = r3_flat.reshape(E, 3)

    g2 = _sc_gather(r8, lg_flat, 2 * L, _G2_WIN)
    g2_flat = g2.reshape(2 * L * D // 128, 128)

    n_half2 = L * D // 128
    b2 = 2000
    cos_flat = pl.pallas_call(
        _cosine_kernel,
        grid=(n_half2 // b2,),
        in_specs=[
            pl.BlockSpec((b2, 128), lambda i: (i, 0)),
            pl.BlockSpec((b2, 128), lambda i: (i + n_half2 // b2, 0)),
        ],
        out_specs=pl.BlockSpec((b2, _RPF), lambda i: (i, 0)),
        out_shape=jax.ShapeDtypeStruct((n_half2, _RPF), jnp.float32),
    )(g2_flat, g2_flat)
    bond_cosine = cos_flat.reshape(L)

    b3 = 1000
    atom_features = pl.pallas_call(
        _embed_kernel,
        grid=(N // b3,),
        in_specs=[
            pl.BlockSpec((b3, 1), lambda i: (i, 0)),
            pl.BlockSpec((128, 128), lambda i: (0, 0)),
        ],
        out_specs=pl.BlockSpec((b3, FEAT_DIM), lambda i: (i, 0)),
        out_shape=jax.ShapeDtypeStruct((N, FEAT_DIM), jnp.float32),
    )(an_col, tab128)

    return (atom_features, r, bond_cosine)


# speedup vs baseline: 8.1849x; 8.1849x over previous
import functools

import jax
import jax.numpy as jnp
from jax.experimental import pallas as pl
from jax.experimental.pallas import tpu as pltpu
from jax.experimental.pallas import tpu_sc as plsc

N = 50000
E = 800000
L = 3200000
NUM_ELEMENTS = 92
FEAT_DIM = 92

D = 8
_RPF = 128 // D
_C3 = 3 * 128 // D

_G1_WIN = 2560
_G2_WIN = 3200


def _sc_gather(table, idx_flat, num_idx, window):
    mesh = plsc.VectorSubcoreMesh(core_axis_name="c", subcore_axis_name="s")

    @functools.partial(
        pl.kernel,
        out_type=jax.ShapeDtypeStruct((num_idx, D), jnp.float32),
        mesh=mesh,
        compiler_params=pltpu.CompilerParams(use_tc_tiling_on_sc=False),
    )
    def gather_kernel(tab_hbm, idx_hbm, out_hbm):
        def body(idx_vmem, out_vmem):
            pltpu.sync_copy(tab_hbm.at[idx_vmem.at[0]], out_vmem)

        pltpu.emit_pipeline(
            body,
            grid=(num_idx // window,),
            in_specs=[pl.BlockSpec((1, window), lambda i: (0, i))],
            out_specs=[pl.BlockSpec((window, D), lambda i: (i, 0))],
            core_axis_name=("c", "s"),
            dimension_semantics=(pltpu.PARALLEL,),
        )(idx_hbm, out_hbm)

    return gather_kernel(table, idx_flat)


def _edge_diff_kernel(src_ref, dst_ref, r8_ref, r3_ref):
    diff = dst_ref[...] - src_ref[...]
    r8_ref[...] = diff
    i = jax.lax.broadcasted_iota(jnp.int32, (128, _C3), 0)
    j = jax.lax.broadcasted_iota(jnp.int32, (128, _C3), 1)
    m = ((i % D < 3) & (j == 3 * (i // D) + (i % D))).astype(jnp.float32)
    r3_ref[...] = jnp.dot(diff, m, preferred_element_type=jnp.float32)


def _cosine_kernel(a_ref, b_ref, out_ref):
    a = a_ref[...]
    b = b_ref[...]
    i = jax.lax.broadcasted_iota(jnp.int32, (128, _RPF), 0)
    j = jax.lax.broadcasted_iota(jnp.int32, (128, _RPF), 1)
    m = (i // D == j).astype(jnp.float32)
    num = -jnp.dot(a * b, m, preferred_element_type=jnp.float32)
    s1 = jnp.dot(a * a, m, preferred_element_type=jnp.float32)
    s2 = jnp.dot(b * b, m, preferred_element_type=jnp.float32)
    denom = jnp.sqrt(s1 * s2)
    out_ref[...] = jnp.clip(num / (denom + 1e-12), -1.0, 1.0)


def _embed_kernel(an_ref, tab_ref, out_ref):
    an = an_ref[...]
    iota = jax.lax.broadcasted_iota(jnp.int32, (an.shape[0], 128), 1)
    oh = (an == iota).astype(jnp.float32)
    res = jnp.dot(oh, tab_ref[...], preferred_element_type=jnp.float32)
    out_ref[...] = res[:, :FEAT_DIM]


def kernel(atomic_number, positions, edge_index, lg_pairs, atom_table):
    pos8 = jnp.pad(positions, ((0, 0), (0, D - 3)))
    eidx_flat = edge_index.reshape(1, 2 * E)
    lg_flat = lg_pairs.reshape(1, 2 * L)
    tab128 = jnp.pad(atom_table, ((0, 128 - NUM_ELEMENTS), (0, 128 - FEAT_DIM)))
    an_col = atomic_number.reshape(N, 1).astype(jnp.int32)

    g1 = _sc_gather(pos8, eidx_flat, 2 * E, _G1_WIN)
    g1_flat = g1.reshape(2 * E * D // 128, 128)

    n_half1 = E * D // 128
    b1 = 1000
    r8_flat, r3_flat = pl.pallas_call(
        _edge_diff_kernel,
        grid=(n_half1 // b1,),
        in_specs=[
            pl.BlockSpec((b1, 128), lambda i: (i, 0)),
            pl.BlockSpec((b1, 128), lambda i: (i + n_half1 // b1, 0)),
        ],
        out_specs=[
            pl.BlockSpec((b1, 128), lambda i: (i, 0)),
            pl.BlockSpec((b1, _C3), lambda i: (i, 0)),
        ],
        out_shape=[
            jax.ShapeDtypeStruct((n_half1, 128), jnp.float32),
            jax.ShapeDtypeStruct((n_half1, _C3), jnp.float32),
        ],
    )(g1_flat, g1_flat)
    r8 = r8_flat.reshape(E, D)
    r = r3_flat.reshape(E, 3)

    g2 = _sc_gather(r8, lg_flat, 2 * L, _G2_WIN)
    g2_flat = g2.reshape(2 * L * D // 128, 128)

    n_half2 = L * D // 128
    b2 = 2000
    cos_flat = pl.pallas_call(
        _cosine_kernel,
        grid=(n_half2 // b2,),
        in_specs=[
            pl.BlockSpec((b2, 128), lambda i: (i, 0)),
            pl.BlockSpec((b2, 128), lambda i: (i + n_half2 // b2, 0)),
        ],
        out_specs=pl.BlockSpec((b2, _RPF), lambda i: (i, 0)),
        out_shape=jax.ShapeDtypeStruct((n_half2, _RPF), jnp.float32),
    )(g2_flat, g2_flat)
    bond_cosine = cos_flat.reshape(L)

    b3 = 1000
    atom_features = pl.pallas_call(
        _embed_kernel,
        grid=(N // b3,),
        in_specs=[
            pl.BlockSpec((b3, 1), lambda i: (i, 0)),
            pl.BlockSpec((128, 128), lambda i: (0, 0)),
        ],
        out_specs=pl.BlockSpec((b3, FEAT_DIM), lambda i: (i, 0)),
        out_shape=jax.ShapeDtypeStruct((N, FEAT_DIM), jnp.float32),
    )(an_col, tab128)

    return (atom_features, r, bond_cosine)
